# Initial kernel scaffold; baseline (speedup 1.0000x reference)
import jax
import jax.numpy as jnp
from jax.experimental import pallas as pl


def kernel(input_ids, decoder_input_ids, table):
    raise NotImplementedError("write your pallas kernel here")


# speedup vs baseline: 1.3724x; 1.3724x over previous
import functools

import jax
import jax.numpy as jnp
from jax import lax
from jax.experimental import pallas as pl
from jax.experimental.pallas import tpu as pltpu
from jax.experimental.pallas import tpu_sc as plsc

_NUM_CORES = 2
_NUM_SUBCORES = 16
_NUM_WORKERS = _NUM_CORES * _NUM_SUBCORES

_CHUNK = 64


def _make_gather(total_rows: int, d_model: int, dtype):
    rows_per_w = total_rows // _NUM_WORKERS
    n_chunks = rows_per_w // _CHUNK
    assert rows_per_w % _CHUNK == 0
    mesh = plsc.VectorSubcoreMesh(
        core_axis_name="c", subcore_axis_name="s",
        num_cores=_NUM_CORES, num_subcores=_NUM_SUBCORES,
    )

    @functools.partial(
        pl.kernel,
        out_type=jax.ShapeDtypeStruct((total_rows, d_model), dtype),
        mesh=mesh,
        scratch_types=[
            pltpu.VMEM((2, _CHUNK), jnp.int32),
            pltpu.VMEM((2, _CHUNK, d_model), dtype),
            pltpu.SemaphoreType.DMA,
            pltpu.SemaphoreType.DMA,
        ],
    )
    def gather_kernel(idx_hbm, table_hbm, out_hbm, idx_v, rows_v, gsem, osem):
        wid = lax.axis_index("s") * _NUM_CORES + lax.axis_index("c")
        base = wid * rows_per_w

        pltpu.sync_copy(idx_hbm.at[pl.ds(base, _CHUNK)], idx_v.at[0])
        pltpu.async_copy(table_hbm.at[idx_v.at[0]], rows_v.at[0], gsem)

        for i in range(n_chunks):
            slot = i % 2
            nxt = (i + 1) % 2
            if i + 1 < n_chunks:
                if i >= 1:
                    pltpu.make_async_copy(
                        rows_v.at[nxt],
                        out_hbm.at[pl.ds(base + (i - 1) * _CHUNK, _CHUNK)],
                        osem,
                    ).wait()
                start = base + (i + 1) * _CHUNK
                pltpu.sync_copy(idx_hbm.at[pl.ds(start, _CHUNK)], idx_v.at[nxt])
                pltpu.async_copy(table_hbm.at[idx_v.at[nxt]], rows_v.at[nxt], gsem)
            pltpu.make_async_copy(
                table_hbm.at[idx_v.at[slot]], rows_v.at[slot], gsem
            ).wait()
            pltpu.async_copy(
                rows_v.at[slot], out_hbm.at[pl.ds(base + i * _CHUNK, _CHUNK)], osem
            )

        for i in (n_chunks - 2, n_chunks - 1):
            pltpu.make_async_copy(
                rows_v.at[i % 2], out_hbm.at[pl.ds(base + i * _CHUNK, _CHUNK)], osem
            ).wait()

    return gather_kernel


@jax.jit
def kernel(input_ids, decoder_input_ids, table):
    b, s = input_ids.shape
    d = table.shape[1]
    idx = jnp.concatenate(
        [input_ids.reshape(-1), decoder_input_ids.reshape(-1)]
    ).astype(jnp.int32)
    out = _make_gather(idx.shape[0], d, table.dtype)(idx, table)
    enc = out[: b * s].reshape(b, s, d)
    dec = out[b * s :].reshape(b, s, d)
    return (enc, dec)


# speedup vs baseline: 1.3769x; 1.0033x over previous
import functools

import jax
import jax.numpy as jnp
from jax import lax
from jax.experimental import pallas as pl
from jax.experimental.pallas import tpu as pltpu
from jax.experimental.pallas import tpu_sc as plsc

_NUM_CORES = 2
_NUM_SUBCORES = 16
_NUM_WORKERS = _NUM_CORES * _NUM_SUBCORES

_CHUNK = 32
_NBUF = 4


def _make_gather(total_rows: int, d_model: int, dtype):
    rows_per_w = total_rows // _NUM_WORKERS
    n_chunks = rows_per_w // _CHUNK
    assert rows_per_w % _CHUNK == 0 and n_chunks >= _NBUF
    mesh = plsc.VectorSubcoreMesh(
        core_axis_name="c", subcore_axis_name="s",
        num_cores=_NUM_CORES, num_subcores=_NUM_SUBCORES,
    )

    @functools.partial(
        pl.kernel,
        out_type=jax.ShapeDtypeStruct((total_rows, d_model), dtype),
        mesh=mesh,
        scratch_types=[
            pltpu.VMEM((rows_per_w,), jnp.int32),
            pltpu.VMEM((_NBUF, _CHUNK, d_model), dtype),
            pltpu.SemaphoreType.DMA,
            pltpu.SemaphoreType.DMA,
        ],
    )
    def gather_kernel(idx_hbm, table_hbm, out_hbm, idx_v, rows_v, gsem, osem):
        wid = lax.axis_index("s") * _NUM_CORES + lax.axis_index("c")
        base = wid * rows_per_w

        def idx_slice(i):
            return idx_v.at[pl.ds(i * _CHUNK, _CHUNK)]

        def out_slice(i):
            return out_hbm.at[pl.ds(base + i * _CHUNK, _CHUNK)]

        pltpu.sync_copy(idx_hbm.at[pl.ds(base, rows_per_w)], idx_v)
        for j in range(_NBUF):
            pltpu.async_copy(table_hbm.at[idx_slice(j)], rows_v.at[j], gsem)

        for i in range(n_chunks):
            buf = i % _NBUF
            if i > 0 and (i - 1) + _NBUF < n_chunks:
                pbuf = (i - 1) % _NBUF
                pltpu.make_async_copy(rows_v.at[pbuf], out_slice(i - 1), osem).wait()
                pltpu.async_copy(
                    table_hbm.at[idx_slice(i - 1 + _NBUF)], rows_v.at[pbuf], gsem
                )
            pltpu.make_async_copy(
                table_hbm.at[idx_slice(i)], rows_v.at[buf], gsem
            ).wait()
            pltpu.async_copy(rows_v.at[buf], out_slice(i), osem)

        for i in range(n_chunks - _NBUF, n_chunks):
            pltpu.make_async_copy(
                rows_v.at[i % _NBUF], out_slice(i), osem
            ).wait()

    return gather_kernel


@jax.jit
def kernel(input_ids, decoder_input_ids, table):
    b, s = input_ids.shape
    d = table.shape[1]
    idx = jnp.concatenate(
        [input_ids.reshape(-1), decoder_input_ids.reshape(-1)]
    ).astype(jnp.int32)
    out = _make_gather(idx.shape[0], d, table.dtype)(idx, table)
    enc = out[: b * s].reshape(b, s, d)
    dec = out[b * s :].reshape(b, s, d)
    return (enc, dec)


# speedup vs baseline: 2.1268x; 1.5446x over previous
import functools

import jax
import jax.numpy as jnp
from jax import lax
from jax.experimental import pallas as pl
from jax.experimental.pallas import tpu as pltpu
from jax.experimental.pallas import tpu_sc as plsc

_NUM_CORES = 2
_NUM_SUBCORES = 16
_NUM_WORKERS = _NUM_CORES * _NUM_SUBCORES

_CHUNK = 32
_NBUF = 4


def _make_gather(n_rows: int, d_model: int, dtype):
    half = _NUM_WORKERS // 2
    rows_per_w = n_rows // half
    n_chunks = rows_per_w // _CHUNK
    assert n_rows % half == 0 and rows_per_w % _CHUNK == 0 and n_chunks >= _NBUF
    mesh = plsc.VectorSubcoreMesh(
        core_axis_name="c", subcore_axis_name="s",
        num_cores=_NUM_CORES, num_subcores=_NUM_SUBCORES,
    )
    out = jax.ShapeDtypeStruct((n_rows, d_model), dtype)

    @functools.partial(
        pl.kernel,
        out_type=(out, out),
        mesh=mesh,
        scratch_types=[
            pltpu.VMEM((rows_per_w,), jnp.int32),
            pltpu.VMEM((_NBUF, _CHUNK, d_model), dtype),
            pltpu.SemaphoreType.DMA,
            pltpu.SemaphoreType.DMA,
        ],
    )
    def gather_kernel(enc_hbm, dec_hbm, table_hbm, enc_out, dec_out,
                      idx_v, rows_v, gsem, osem):
        wid = lax.axis_index("s") * _NUM_CORES + lax.axis_index("c")

        def run(idx_hbm, out_hbm, slot):
            base = slot * rows_per_w

            def idx_slice(i):
                return idx_v.at[pl.ds(i * _CHUNK, _CHUNK)]

            def out_slice(i):
                return out_hbm.at[pl.ds(base + i * _CHUNK, _CHUNK)]

            pltpu.sync_copy(idx_hbm.at[pl.ds(base, rows_per_w)], idx_v)
            for j in range(_NBUF):
                pltpu.async_copy(table_hbm.at[idx_slice(j)], rows_v.at[j], gsem)

            for i in range(n_chunks):
                buf = i % _NBUF
                if i > 0 and (i - 1) + _NBUF < n_chunks:
                    pbuf = (i - 1) % _NBUF
                    pltpu.make_async_copy(
                        rows_v.at[pbuf], out_slice(i - 1), osem
                    ).wait()
                    pltpu.async_copy(
                        table_hbm.at[idx_slice(i - 1 + _NBUF)], rows_v.at[pbuf], gsem
                    )
                pltpu.make_async_copy(
                    table_hbm.at[idx_slice(i)], rows_v.at[buf], gsem
                ).wait()
                pltpu.async_copy(rows_v.at[buf], out_slice(i), osem)

            for i in range(n_chunks - _NBUF, n_chunks):
                pltpu.make_async_copy(
                    rows_v.at[i % _NBUF], out_slice(i), osem
                ).wait()

        @pl.when(wid < half)
        def _():
            run(enc_hbm, enc_out, wid)

        @pl.when(wid >= half)
        def _():
            run(dec_hbm, dec_out, wid - half)

    return gather_kernel


@jax.jit
def kernel(input_ids, decoder_input_ids, table):
    b, s = input_ids.shape
    d = table.shape[1]
    enc_ids = input_ids.reshape(-1).astype(jnp.int32)
    dec_ids = decoder_input_ids.reshape(-1).astype(jnp.int32)
    enc, dec = _make_gather(b * s, d, table.dtype)(enc_ids, dec_ids, table)
    return (enc.reshape(b, s, d), dec.reshape(b, s, d))


# speedup vs baseline: 2.1375x; 1.0050x over previous
import functools

import jax
import jax.numpy as jnp
from jax import lax
from jax.experimental import pallas as pl
from jax.experimental.pallas import tpu as pltpu
from jax.experimental.pallas import tpu_sc as plsc

_NUM_CORES = 2
_NUM_SUBCORES = 16
_NUM_WORKERS = _NUM_CORES * _NUM_SUBCORES

_CHUNK = 32
_NBUF = 4


def _make_gather(batch: int, seq: int, d_model: int, dtype):
    half = _NUM_WORKERS // 2
    n_rows = batch * seq
    rows_per_w = n_rows // half
    n_chunks = rows_per_w // _CHUNK
    assert n_rows % half == 0 and rows_per_w % _CHUNK == 0 and n_chunks >= _NBUF
    assert seq % rows_per_w == 0 or rows_per_w % seq == 0
    mesh = plsc.VectorSubcoreMesh(
        core_axis_name="c", subcore_axis_name="s",
        num_cores=_NUM_CORES, num_subcores=_NUM_SUBCORES,
    )
    out = jax.ShapeDtypeStruct((batch, seq, d_model), dtype)

    @functools.partial(
        pl.kernel,
        out_type=(out, out),
        mesh=mesh,
        scratch_types=[
            pltpu.VMEM((rows_per_w,), jnp.int32),
            pltpu.VMEM((_NBUF, _CHUNK, d_model), dtype),
            pltpu.SemaphoreType.DMA,
            pltpu.SemaphoreType.DMA,
        ],
    )
    def gather_kernel(enc_hbm, dec_hbm, table_hbm, enc_out, dec_out,
                      idx_v, rows_v, gsem, osem):
        wid = lax.axis_index("s") * _NUM_CORES + lax.axis_index("c")

        def run(idx_hbm, out_hbm, slot):
            b = (slot * rows_per_w) // seq
            t0 = (slot * rows_per_w) % seq

            def idx_slice(i):
                return idx_v.at[pl.ds(i * _CHUNK, _CHUNK)]

            def out_slice(i):
                return out_hbm.at[b, pl.ds(t0 + i * _CHUNK, _CHUNK), :]

            pltpu.sync_copy(idx_hbm.at[b, pl.ds(t0, rows_per_w)], idx_v)
            for j in range(_NBUF):
                pltpu.async_copy(table_hbm.at[idx_slice(j)], rows_v.at[j], gsem)

            for i in range(n_chunks):
                buf = i % _NBUF
                if i > 0 and (i - 1) + _NBUF < n_chunks:
                    pbuf = (i - 1) % _NBUF
                    pltpu.make_async_copy(
                        rows_v.at[pbuf], out_slice(i - 1), osem
                    ).wait()
                    pltpu.async_copy(
                        table_hbm.at[idx_slice(i - 1 + _NBUF)], rows_v.at[pbuf], gsem
                    )
                pltpu.make_async_copy(
                    table_hbm.at[idx_slice(i)], rows_v.at[buf], gsem
                ).wait()
                pltpu.async_copy(rows_v.at[buf], out_slice(i), osem)

            for i in range(n_chunks - _NBUF, n_chunks):
                pltpu.make_async_copy(
                    rows_v.at[i % _NBUF], out_slice(i), osem
                ).wait()

        @pl.when(wid < half)
        def _():
            run(enc_hbm, enc_out, wid)

        @pl.when(wid >= half)
        def _():
            run(dec_hbm, dec_out, wid - half)

    return gather_kernel


@jax.jit
def kernel(input_ids, decoder_input_ids, table):
    b, s = input_ids.shape
    d = table.shape[1]
    return _make_gather(b, s, d, table.dtype)(
        input_ids.astype(jnp.int32), decoder_input_ids.astype(jnp.int32), table
    )
